# baseline (speedup 1.0000x reference)
import functools

import jax
import jax.numpy as jnp
from jax import lax
from jax.experimental import pallas as pl
from jax.experimental.pallas import tpu as pltpu
from jax.experimental.pallas import tpu_sc as plsc

_LANES = 16
_CHUNK = 8
_NBUF = 4
_AHEAD = 3


@functools.lru_cache(maxsize=None)
def _build(n_batch, t_cur, d_model):
    info = plsc.get_sparse_core_info()
    nc, ns = info.num_cores, info.num_subcores
    nw = nc * ns
    tpw = t_cur // nw
    c = _CHUNK
    nch_b = tpw // c
    nch = n_batch * nch_b
    cw = d_model // _LANES
    n_rows = n_batch * t_cur

    mesh = plsc.VectorSubcoreMesh(core_axis_name="c", subcore_axis_name="s")

    @functools.partial(
        pl.kernel,
        mesh=mesh,
        out_type=jax.ShapeDtypeStruct((n_rows, d_model), jnp.float32),
        scratch_types=[
            pltpu.VMEM((n_batch * tpw,), jnp.int32),
            pltpu.VMEM((tpw, d_model), jnp.float32),
            pltpu.VMEM((_NBUF, c, d_model), jnp.float32),
            pltpu.SemaphoreType.DMA,
            pltpu.SemaphoreType.DMA,
            pltpu.SemaphoreType.DMA,
            pltpu.SemaphoreType.DMA,
            pltpu.SemaphoreType.DMA,
            pltpu.SemaphoreType.DMA,
            pltpu.SemaphoreType.DMA,
            pltpu.SemaphoreType.DMA,
            pltpu.SemaphoreType.DMA,
            pltpu.SemaphoreType.DMA,
        ],
    )
    def emb_kernel(idx_hbm, tok_hbm, pos_hbm, out_hbm,
                   idx_v, pos_v, rows_v, si, sp,
                   sg0, sg1, sg2, sg3, ss0, ss1, ss2, ss3):
        wid = lax.axis_index("s") * nc + lax.axis_index("c")
        tbase = wid * tpw

        pos_cp = pltpu.async_copy(pos_hbm.at[pl.ds(tbase, tpw)], pos_v, sp)
        icp = [pltpu.async_copy(idx_hbm.at[pl.ds(b * t_cur + tbase, tpw)],
                                idx_v.at[pl.ds(b * tpw, tpw)], si)
               for b in range(n_batch)]

        sgs = (sg0, sg1, sg2, sg3)
        sss = (ss0, ss1, ss2, ss3)
        gcp = [None] * _NBUF
        scp = [None] * _NBUF
        idx_ready = [False] * n_batch

        def issue(g):
            b = g // nch_b
            if not idx_ready[b]:
                icp[b].wait()
                idx_ready[b] = True
            buf = g % _NBUF
            if scp[buf] is not None:
                scp[buf].wait()
            gcp[buf] = pltpu.async_copy(
                tok_hbm.at[idx_v.at[pl.ds(g * c, c)]], rows_v.at[buf], sgs[buf])

        for g in range(min(_AHEAD + 1, nch)):
            issue(g)
        pos_cp.wait()

        for g in range(nch):
            buf = g % _NBUF
            b, q = divmod(g, nch_b)
            gcp[buf].wait()

            def _make_add(_buf, _pr):
                @plsc.parallel_loop(0, c * cw, unroll=16)
                def add_body(i):
                    r = i // cw
                    col = (i % cw) * _LANES
                    x = pos_v[_pr + r, pl.ds(col, _LANES)]
                    plsc.addupdate(rows_v.at[_buf, r, pl.ds(col, _LANES)], x)

            _make_add(buf, q * c)
            out_row = b * t_cur + tbase + q * c
            scp[buf] = pltpu.async_copy(
                rows_v.at[buf], out_hbm.at[pl.ds(out_row, c)], sss[buf])
            if g + _AHEAD + 1 < nch:
                issue(g + _AHEAD + 1)

        for buf in range(_NBUF):
            if scp[buf] is not None:
                scp[buf].wait()

    return emb_kernel


def kernel(idx, tok_emb, pos_emb):
    n_batch, t_cur = idx.shape
    d_model = tok_emb.shape[1]
    flat_idx = idx.reshape(n_batch * t_cur).astype(jnp.int32)
    out = _build(n_batch, t_cur, d_model)(flat_idx, tok_emb, pos_emb)
    return out.reshape(n_batch, t_cur, d_model)


# speedup vs baseline: 1.0451x; 1.0451x over previous
import functools

import jax
import jax.numpy as jnp
from jax import lax
from jax.experimental import pallas as pl
from jax.experimental.pallas import tpu as pltpu
from jax.experimental.pallas import tpu_sc as plsc

_LANES = 16
_CHUNK = 8
_NBUF = 4
_AHEAD = 3


@functools.lru_cache(maxsize=None)
def _build(n_batch, t_cur, d_model):
    info = plsc.get_sparse_core_info()
    nc, ns = info.num_cores, info.num_subcores
    nw = nc * ns
    tpw = t_cur // nw
    c = _CHUNK
    nch_b = tpw // c
    nch = n_batch * nch_b
    cw = d_model // _LANES
    n_rows = n_batch * t_cur

    mesh = plsc.VectorSubcoreMesh(core_axis_name="c", subcore_axis_name="s")

    @functools.partial(
        pl.kernel,
        mesh=mesh,
        out_type=jax.ShapeDtypeStruct((n_rows, d_model), jnp.float32),
        scratch_types=[
            pltpu.VMEM((n_batch * tpw,), jnp.int32),
            pltpu.VMEM((tpw, d_model), jnp.float32),
            pltpu.VMEM((_NBUF, c, d_model), jnp.float32),
            pltpu.SemaphoreType.DMA,
            pltpu.SemaphoreType.DMA,
            pltpu.SemaphoreType.DMA,
            pltpu.SemaphoreType.DMA,
            pltpu.SemaphoreType.DMA,
            pltpu.SemaphoreType.DMA,
            pltpu.SemaphoreType.DMA,
            pltpu.SemaphoreType.DMA,
            pltpu.SemaphoreType.DMA,
            pltpu.SemaphoreType.DMA,
        ],
    )
    def emb_kernel(idx_hbm, tok_hbm, pos_hbm, out_hbm,
                   idx_v, pos_v, rows_v, si, sp,
                   sg0, sg1, sg2, sg3, ss0, ss1, ss2, ss3):
        wid = lax.axis_index("s") * nc + lax.axis_index("c")
        tbase = wid * tpw

        pos_cp = pltpu.async_copy(pos_hbm.at[pl.ds(tbase, tpw)], pos_v, sp)
        icp = [pltpu.async_copy(idx_hbm.at[pl.ds(b * t_cur + tbase, tpw)],
                                idx_v.at[pl.ds(b * tpw, tpw)], si)
               for b in range(n_batch)]

        sgs = (sg0, sg1, sg2, sg3)
        sss = (ss0, ss1, ss2, ss3)
        gcp = [None] * _NBUF
        scp = [None] * _NBUF
        idx_ready = [False] * n_batch

        def issue(g):
            b = g // nch_b
            if not idx_ready[b]:
                icp[b].wait()
                idx_ready[b] = True
            buf = g % _NBUF
            if scp[buf] is not None:
                scp[buf].wait()
            gcp[buf] = pltpu.async_copy(
                tok_hbm.at[idx_v.at[pl.ds(g * c, c)]], rows_v.at[buf], sgs[buf])

        for g in range(min(_AHEAD + 1, nch)):
            issue(g)
        pos_cp.wait()

        for g in range(nch):
            buf = g % _NBUF
            b, q = divmod(g, nch_b)
            gcp[buf].wait()

            def _make_add(_buf, _pr):
                @plsc.parallel_loop(0, c * cw, unroll=8)
                def add_body(i):
                    r = i // cw
                    col = (i % cw) * _LANES
                    x = pos_v[_pr + r, pl.ds(col, _LANES)]
                    plsc.addupdate(rows_v.at[_buf, r, pl.ds(col, _LANES)], x)

            _make_add(buf, q * c)
            out_row = b * t_cur + tbase + q * c
            scp[buf] = pltpu.async_copy(
                rows_v.at[buf], out_hbm.at[pl.ds(out_row, c)], sss[buf])
            if g + _AHEAD + 1 < nch:
                issue(g + _AHEAD + 1)

        for buf in range(_NBUF):
            if scp[buf] is not None:
                scp[buf].wait()

    return emb_kernel


def kernel(idx, tok_emb, pos_emb):
    n_batch, t_cur = idx.shape
    d_model = tok_emb.shape[1]
    flat_idx = idx.reshape(n_batch * t_cur).astype(jnp.int32)
    out = _build(n_batch, t_cur, d_model)(flat_idx, tok_emb, pos_emb)
    return out.reshape(n_batch, t_cur, d_model)


# speedup vs baseline: 1.8731x; 1.7923x over previous
import functools

import jax
import jax.numpy as jnp
from jax import lax
from jax.experimental import pallas as pl
from jax.experimental.pallas import tpu as pltpu
from jax.experimental.pallas import tpu_sc as plsc

_LANES = 16
_CHUNK = 8
_NBUF = 4
_AHEAD = 2


@functools.lru_cache(maxsize=None)
def _build(n_batch, t_cur, d_model):
    info = plsc.get_sparse_core_info()
    nc, ns = info.num_cores, info.num_subcores
    nw = nc * ns
    tpw = t_cur // nw
    c = _CHUNK
    nch_b = tpw // c
    nch = n_batch * nch_b
    cw = d_model // _LANES
    n_rows = n_batch * t_cur

    mesh = plsc.VectorSubcoreMesh(core_axis_name="c", subcore_axis_name="s")

    @functools.partial(
        pl.kernel,
        mesh=mesh,
        out_type=jax.ShapeDtypeStruct((n_rows, d_model), jnp.float32),
        scratch_types=[
            pltpu.VMEM((n_batch * tpw,), jnp.int32),
            pltpu.VMEM((tpw, d_model), jnp.float32),
            pltpu.VMEM((_NBUF, c, d_model), jnp.float32),
            pltpu.SemaphoreType.DMA,
            pltpu.SemaphoreType.DMA,
            pltpu.SemaphoreType.DMA,
            pltpu.SemaphoreType.DMA,
            pltpu.SemaphoreType.DMA,
            pltpu.SemaphoreType.DMA,
            pltpu.SemaphoreType.DMA,
            pltpu.SemaphoreType.DMA,
            pltpu.SemaphoreType.DMA,
            pltpu.SemaphoreType.DMA,
        ],
    )
    def emb_kernel(idx_hbm, tok_hbm, pos_hbm, out_hbm,
                   idx_v, pos_v, rows_v, si, sp,
                   sg0, sg1, sg2, sg3, ss0, ss1, ss2, ss3):
        wid = lax.axis_index("s") * nc + lax.axis_index("c")
        tbase = wid * tpw

        pos_cp = pltpu.async_copy(pos_hbm.at[pl.ds(tbase, tpw)], pos_v, sp)
        icp = [pltpu.async_copy(idx_hbm.at[pl.ds(b * t_cur + tbase, tpw)],
                                idx_v.at[pl.ds(b * tpw, tpw)], si)
               for b in range(n_batch)]

        sgs = (sg0, sg1, sg2, sg3)
        sss = (ss0, ss1, ss2, ss3)
        gcp = [None] * _NBUF
        scp = [None] * _NBUF
        idx_ready = [False] * n_batch

        def issue(g):
            b = g // nch_b
            if not idx_ready[b]:
                icp[b].wait()
                idx_ready[b] = True
            buf = g % _NBUF
            if scp[buf] is not None:
                scp[buf].wait()
            gcp[buf] = pltpu.async_copy(
                tok_hbm.at[idx_v.at[pl.ds(g * c, c)]], rows_v.at[buf], sgs[buf])

        for g in range(min(_AHEAD + 1, nch)):
            issue(g)
        pos_cp.wait()

        for g in range(nch):
            buf = g % _NBUF
            b, q = divmod(g, nch_b)
            gcp[buf].wait()

            def _make_add(_buf, _pr):
                @plsc.parallel_loop(0, c * cw, unroll=8)
                def add_body(i):
                    r = i // cw
                    col = (i % cw) * _LANES
                    x = pos_v[_pr + r, pl.ds(col, _LANES)]
                    plsc.addupdate(rows_v.at[_buf, r, pl.ds(col, _LANES)], x)

            _make_add(buf, q * c)
            out_row = b * t_cur + tbase + q * c
            scp[buf] = pltpu.async_copy(
                rows_v.at[buf], out_hbm.at[pl.ds(out_row, c)], sss[buf])
            if g + _AHEAD + 1 < nch:
                issue(g + _AHEAD + 1)

        for buf in range(_NBUF):
            if scp[buf] is not None:
                scp[buf].wait()

    return emb_kernel


def kernel(idx, tok_emb, pos_emb):
    n_batch, t_cur = idx.shape
    d_model = tok_emb.shape[1]
    flat_idx = idx.reshape(n_batch * t_cur).astype(jnp.int32)
    out = _build(n_batch, t_cur, d_model)(flat_idx, tok_emb, pos_emb)
    return out.reshape(n_batch, t_cur, d_model)
